# Initial kernel scaffold; baseline (speedup 1.0000x reference)
import jax
import jax.numpy as jnp
from jax.experimental import pallas as pl


def kernel(batch_node_pred, batch_node_input, batch_edge_input, rainfall, node_mean, node_std, edge_mean, edge_std, edge_index, batch, node_filter_mask):
    raise NotImplementedError("write your pallas kernel here")


# speedup vs baseline: 27.0711x; 27.0711x over previous
import functools

import jax
import jax.numpy as jnp
from jax import lax
from jax.experimental import pallas as pl
from jax.experimental.pallas import tpu as pltpu
from jax.experimental.pallas import tpu_sc as plsc

_N = 50000
_E = 1600000
_NW = 32
_EPT = _E // _NW
_C = 10000
_NCH = _EPT // _C
_DT = 30.0
_NG = 16.0


def _sc_scatter_build():
    mesh = plsc.VectorSubcoreMesh(core_axis_name="c", subcore_axis_name="s")

    @functools.partial(
        pl.kernel,
        mesh=mesh,
        out_type=jax.ShapeDtypeStruct((_NW, _N), jnp.float32),
        compiler_params=pltpu.CompilerParams(needs_layout_passes=False),
        scratch_types=[
            pltpu.VMEM((_N,), jnp.float32),
            pltpu.VMEM((_C,), jnp.int32),
            pltpu.VMEM((_C,), jnp.int32),
            pltpu.VMEM((_C,), jnp.float32),
            pltpu.VMEM((32,), jnp.float32),
        ],
    )
    def sc_scatter(row_hbm, col_hbm, e0_hbm, cst_hbm, out_hbm,
                   acc, rowv, colv, flowv, cstv):
        wid = lax.axis_index("s") * 2 + lax.axis_index("c")
        base = wid * _EPT
        pltpu.sync_copy(cst_hbm, cstv)
        scale = cstv[pl.ds(0, 16)]
        mean = cstv[pl.ds(16, 16)]
        zero = jnp.zeros((16,), jnp.float32)

        def zbody(i, carry):
            acc[pl.ds(i * 16, 16)] = zero
            return carry

        lax.fori_loop(0, _N // 16, zbody, 0)

        def chunk(ci, carry):
            off = base + ci * _C
            pltpu.sync_copy(row_hbm.at[pl.ds(off, _C)], rowv)
            pltpu.sync_copy(col_hbm.at[pl.ds(off, _C)], colv)
            pltpu.sync_copy(e0_hbm.at[pl.ds(off, _C)], flowv)

            def ebody(i, c2):
                sl = pl.ds(i * 16, 16)
                f = flowv[sl] * scale + mean
                plsc.addupdate_scatter(acc, [colv[sl]], f)
                plsc.addupdate_scatter(acc, [rowv[sl]], -f)
                return c2

            lax.fori_loop(0, _C // 16, ebody, carry)
            return carry

        lax.fori_loop(0, _NCH, chunk, 0)
        pltpu.sync_copy(acc, out_hbm.at[wid])

    return sc_scatter


_sc_scatter = _sc_scatter_build()


def _fin_body(part_ref, p0_ref, i0_ref, rain_ref, mask_ref, std_ref, out_ref):
    net = jnp.sum(part_ref[...], axis=0)
    d = (p0_ref[...] - i0_ref[...]) * std_ref[...][0, 0]
    err = d - _DT * net - rain_ref[...]
    tot = jnp.sum(mask_ref[...] * jnp.abs(err))
    out_ref[...] = (tot / _NG).reshape(1, 1)


def _finalize(partials, p0, i0, rain, maskf, std):
    return pl.pallas_call(
        _fin_body,
        out_shape=jax.ShapeDtypeStruct((1, 1), jnp.float32),
    )(partials, p0, i0, rain, maskf, std)


def kernel(batch_node_pred, batch_node_input, batch_edge_input, rainfall,
           node_mean, node_std, edge_mean, edge_std,
           edge_index, batch, node_filter_mask):
    row = edge_index[0]
    col = edge_index[1]
    e0 = batch_edge_input[:, 0]
    cst = jnp.concatenate([
        jnp.broadcast_to(edge_std[0], (16,)),
        jnp.broadcast_to(edge_mean[0], (16,)),
    ]).astype(jnp.float32)
    partials = _sc_scatter(row, col, e0, cst)

    p0 = batch_node_pred[:, 0].reshape(8, 6250)
    i0 = batch_node_input[:, 0].reshape(8, 6250)
    rain = rainfall.reshape(8, 6250)
    maskf = node_filter_mask.astype(jnp.float32).reshape(8, 6250)
    std = node_std[0].reshape(1, 1)
    loss = _finalize(partials.reshape(32, 8, 6250), p0, i0, rain, maskf, std)
    return loss[0, 0]


# speedup vs baseline: 30.7374x; 1.1354x over previous
import functools

import jax
import jax.numpy as jnp
from jax import lax
from jax.experimental import pallas as pl
from jax.experimental.pallas import tpu as pltpu
from jax.experimental.pallas import tpu_sc as plsc

_N = 50000
_E = 1600000
_NW = 32
_EPT = _E // _NW
_C = 2000
_NCH = _EPT // _C
_DT = 30.0
_NG = 16.0


def _sc_scatter_build():
    mesh = plsc.VectorSubcoreMesh(core_axis_name="c", subcore_axis_name="s")

    @functools.partial(
        pl.kernel,
        mesh=mesh,
        out_type=jax.ShapeDtypeStruct((_NW, _N), jnp.float32),
        compiler_params=pltpu.CompilerParams(needs_layout_passes=False),
        scratch_types=[
            pltpu.VMEM((_N,), jnp.float32),
            pltpu.VMEM((_C,), jnp.int32),
            pltpu.VMEM((_C,), jnp.int32),
            pltpu.VMEM((_C,), jnp.float32),
            pltpu.VMEM((_C,), jnp.int32),
            pltpu.VMEM((_C,), jnp.int32),
            pltpu.VMEM((_C,), jnp.float32),
            pltpu.VMEM((32,), jnp.float32),
            pltpu.SemaphoreType.DMA,
            pltpu.SemaphoreType.DMA,
        ],
    )
    def sc_scatter(row_hbm, col_hbm, e0_hbm, cst_hbm, out_hbm,
                   acc, row0, col0, flow0, row1, col1, flow1, cstv,
                   sem0, sem1):
        wid = lax.axis_index("s") * 2 + lax.axis_index("c")
        base = wid * _EPT
        pltpu.sync_copy(cst_hbm, cstv)
        scale = cstv[pl.ds(0, 16)]
        mean = cstv[pl.ds(16, 16)]
        zero = jnp.zeros((16,), jnp.float32)
        bufs = ((row0, col0, flow0, sem0), (row1, col1, flow1, sem1))

        def zbody(i, carry):
            acc[pl.ds(i * 16, 16)] = zero
            return carry

        lax.fori_loop(0, _N // 16, zbody, 0, unroll=8)

        def start(ci, b):
            rv, cv, fv, sem = bufs[b]
            off = base + ci * _C
            pltpu.async_copy(row_hbm.at[pl.ds(off, _C)], rv, sem)
            pltpu.async_copy(col_hbm.at[pl.ds(off, _C)], cv, sem)
            pltpu.async_copy(e0_hbm.at[pl.ds(off, _C)], fv, sem)

        def drain_and_scatter(b, carry):
            rv, cv, fv, sem = bufs[b]
            pltpu.make_async_copy(row_hbm.at[pl.ds(0, _C)], rv, sem).wait()
            pltpu.make_async_copy(col_hbm.at[pl.ds(0, _C)], cv, sem).wait()
            pltpu.make_async_copy(e0_hbm.at[pl.ds(0, _C)], fv, sem).wait()

            def ebody(i, c2):
                sl = pl.ds(i * 16, 16)
                f = fv[sl] * scale + mean
                plsc.addupdate_scatter(acc, [cv[sl]], f)
                plsc.addupdate_scatter(acc, [rv[sl]], -f)
                return c2

            return lax.fori_loop(0, _C // 16, ebody, carry, unroll=8)

        start(0, 0)

        def pair(pi, carry):
            c = 2 * pi
            start(c + 1, 1)
            carry = drain_and_scatter(0, carry)
            start(c + 2, 0)
            carry = drain_and_scatter(1, carry)
            return carry

        carry = lax.fori_loop(0, (_NCH - 1) // 2, pair, 0)
        drain_and_scatter(0, carry)
        pltpu.sync_copy(acc, out_hbm.at[wid])

    return sc_scatter


@functools.cache
def _sc_scatter():
    return _sc_scatter_build()


def _fin_body(part_ref, p0_ref, i0_ref, rain_ref, mask_ref, std_ref, out_ref):
    net = jnp.sum(part_ref[...], axis=0)
    d = (p0_ref[...] - i0_ref[...]) * std_ref[...][0, 0]
    err = d - _DT * net - rain_ref[...]
    tot = jnp.sum(mask_ref[...] * jnp.abs(err))
    out_ref[...] = (tot / _NG).reshape(1, 1)


def _finalize(partials, p0, i0, rain, maskf, std):
    return pl.pallas_call(
        _fin_body,
        out_shape=jax.ShapeDtypeStruct((1, 1), jnp.float32),
    )(partials, p0, i0, rain, maskf, std)


def kernel(batch_node_pred, batch_node_input, batch_edge_input, rainfall,
           node_mean, node_std, edge_mean, edge_std,
           edge_index, batch, node_filter_mask):
    row = edge_index[0]
    col = edge_index[1]
    e0 = batch_edge_input[:, 0]
    cst = jnp.concatenate([
        jnp.broadcast_to(edge_std[0], (16,)),
        jnp.broadcast_to(edge_mean[0], (16,)),
    ]).astype(jnp.float32)
    partials = _sc_scatter()(row, col, e0, cst)

    p0 = batch_node_pred[:, 0].reshape(8, 6250)
    i0 = batch_node_input[:, 0].reshape(8, 6250)
    rain = rainfall.reshape(8, 6250)
    maskf = node_filter_mask.astype(jnp.float32).reshape(8, 6250)
    std = node_std[0].reshape(1, 1)
    loss = _finalize(partials.reshape(32, 8, 6250), p0, i0, rain, maskf, std)
    return loss[0, 0]
